# baseline (speedup 1.0000x reference)
import functools

import jax
import jax.numpy as jnp
from jax import lax
from jax.experimental import pallas as pl
from jax.experimental.pallas import tpu as pltpu
from jax.experimental.pallas import tpu_sc as plsc

_NUM_WORKERS = 32
_ROWS_PER_DMA = 128
_SUBS = 8
_CHUNK = _SUBS * _ROWS_PER_DMA


@functools.lru_cache(maxsize=None)
def _make_gather(n_total: int, dim: int):
    assert n_total % (_NUM_WORKERS * _CHUNK) == 0
    per_worker = n_total // _NUM_WORKERS
    nchunks = per_worker // _CHUNK
    nsteps = per_worker // _ROWS_PER_DMA

    mesh = plsc.VectorSubcoreMesh(core_axis_name="c", subcore_axis_name="s")

    @functools.partial(
        pl.kernel,
        mesh=mesh,
        out_type=jax.ShapeDtypeStruct((n_total, dim), jnp.float32),
        scratch_types=[
            pltpu.VMEM((nsteps, _ROWS_PER_DMA), jnp.int32),
            pltpu.VMEM((2, _CHUNK, dim), jnp.float32),
            pltpu.SemaphoreType.DMA,
        ],
        compiler_params=pltpu.CompilerParams(use_tc_tiling_on_sc=False),
    )
    def gather_kernel(idx_hbm, table_hbm, out_hbm, idx_v, rows_v, sem):
        wid = lax.axis_index("s") * 2 + lax.axis_index("c")
        base = wid * per_worker
        pltpu.sync_copy(idx_hbm.at[wid], idx_v)

        def fire(g, b):
            for j in range(_SUBS):
                pltpu.async_copy(
                    table_hbm.at[idx_v.at[g * _SUBS + j]],
                    rows_v.at[b, pl.ds(j * _ROWS_PER_DMA, _ROWS_PER_DMA)],
                    sem,
                )

        def drain_and_scatter(g, b):
            for j in range(_SUBS):
                pltpu.make_async_copy(
                    table_hbm.at[idx_v.at[g * _SUBS + j]],
                    rows_v.at[b, pl.ds(j * _ROWS_PER_DMA, _ROWS_PER_DMA)],
                    sem,
                ).wait()
            pltpu.sync_copy(
                rows_v.at[b], out_hbm.at[pl.ds(base + g * _CHUNK, _CHUNK)]
            )

        fire(0, 0)

        def body(g, carry):
            fire(g + 1, lax.rem(g + 1, 2))
            drain_and_scatter(g, lax.rem(g, 2))
            return carry

        lax.fori_loop(0, nchunks - 1, body, 0)
        drain_and_scatter(nchunks - 1, (nchunks - 1) % 2)

    return gather_kernel


_TCH = 256
_TPAD = 128


@functools.lru_cache(maxsize=None)
def _make_sc_transpose(v: int, dim: int):
    v_pad = ((v + _TPAD - 1) // _TPAD) * _TPAD
    nchunks = -(-v // _TCH)
    per_worker = -(-nchunks // _NUM_WORKERS)
    per_worker += per_worker % 2
    groups = per_worker // 2
    max_off = v_pad - _TCH

    mesh = plsc.VectorSubcoreMesh(core_axis_name="c", subcore_axis_name="s")

    @functools.partial(
        pl.kernel,
        mesh=mesh,
        out_type=jax.ShapeDtypeStruct((v_pad * dim,), jnp.float32),
        scratch_types=[
            pltpu.VMEM((dim, _TCH + 5), jnp.float32),
            pltpu.VMEM((dim, _TCH + 5), jnp.float32),
            pltpu.VMEM((_TCH * dim,), jnp.float32),
            pltpu.VMEM((_TCH * dim,), jnp.float32),
            pltpu.SemaphoreType.DMA,
            pltpu.SemaphoreType.DMA,
            pltpu.SemaphoreType.DMA,
            pltpu.SemaphoreType.DMA,
        ],
        compiler_params=pltpu.CompilerParams(
            use_tc_tiling_on_sc=True, needs_layout_passes=False),
    )
    def tr(tbl_hbm, out_hbm, buf_a, buf_b, ob_a, ob_b, s_a, s_b, o_a, o_b):
        wid = lax.axis_index("s") * 2 + lax.axis_index("c")
        first = wid * per_worker

        def col_off(c):
            return jnp.minimum(c * _TCH, max_off)

        def start_in(c, buf, sem):
            pltpu.async_copy(tbl_hbm.at[:, pl.ds(col_off(c), _TCH)],
                             buf.at[:, pl.ds(0, _TCH)], sem)

        def wait_in(c, buf, sem):
            pltpu.make_async_copy(
                tbl_hbm.at[:, pl.ds(col_off(c), _TCH)],
                buf.at[:, pl.ds(0, _TCH)], sem).wait()

        def start_out(c, ob, sem):
            pltpu.async_copy(
                ob, out_hbm.at[pl.ds(col_off(c) * dim, _TCH * dim)], sem)

        def wait_out(c, ob, sem):
            pltpu.make_async_copy(
                ob, out_hbm.at[pl.ds(col_off(c) * dim, _TCH * dim)], sem).wait()

        lanes16 = lax.iota(jnp.int32, 16)
        cols_h = [lanes16 + 16 * h for h in range(dim // 16)]

        def shuffle(buf, ob):
            for r in range(_TCH):
                row = jnp.full((16,), r, jnp.int32)
                for h in range(dim // 16):
                    vals = plsc.load_gather(buf, [cols_h[h], row])
                    ob[pl.ds(r * dim + 16 * h, 16)] = vals

        start_in(first, buf_a, s_a)

        def body(g, carry):
            c0 = first + 2 * g
            c1 = c0 + 1
            start_in(c1, buf_b, s_b)
            wait_in(c0, buf_a, s_a)

            @pl.when(g > 0)
            def _():
                wait_out(c0, ob_a, o_a)

            shuffle(buf_a, ob_a)
            start_out(c0, ob_a, o_a)

            @pl.when(g + 1 < groups)
            def _():
                start_in(c0 + 2, buf_a, s_a)

            wait_in(c1, buf_b, s_b)

            @pl.when(g > 0)
            def _():
                wait_out(c1, ob_b, o_b)

            shuffle(buf_b, ob_b)
            start_out(c1, ob_b, o_b)
            return carry

        lax.fori_loop(0, groups, body, 0)
        wait_out(first, ob_a, o_a)
        wait_out(first, ob_b, o_b)

    return tr


def kernel(indices, table):
    shape = indices.shape
    v, dim = table.shape
    v_pad = ((v + _TPAD - 1) // _TPAD) * _TPAD
    table_rm = _make_sc_transpose(v, dim)(table.T).reshape(v_pad, dim)
    flat = indices.reshape(-1).astype(jnp.int32)
    n = flat.shape[0]
    group = _NUM_WORKERS * _CHUNK
    n_pad = ((n + group - 1) // group) * group
    if n_pad != n:
        flat = jnp.concatenate([flat, jnp.zeros((n_pad - n,), jnp.int32)])
    idx3 = flat.reshape(_NUM_WORKERS, -1, _ROWS_PER_DMA)
    out = _make_gather(n_pad, dim)(idx3, table_rm)
    if n_pad != n:
        out = out[:n]
    return out.reshape(*shape, dim)


# speedup vs baseline: 1.9153x; 1.9153x over previous
import functools

import jax
import jax.numpy as jnp
from jax import lax
from jax.experimental import pallas as pl
from jax.experimental.pallas import tpu as pltpu
from jax.experimental.pallas import tpu_sc as plsc

_NUM_WORKERS = 32
_ROWS_PER_DMA = 128
_SUBS = 8
_CHUNK = _SUBS * _ROWS_PER_DMA


@functools.lru_cache(maxsize=None)
def _make_gather(n_total: int, dim: int):
    assert n_total % (_NUM_WORKERS * _CHUNK) == 0
    per_worker = n_total // _NUM_WORKERS
    nchunks = per_worker // _CHUNK
    nsteps = per_worker // _ROWS_PER_DMA

    mesh = plsc.VectorSubcoreMesh(core_axis_name="c", subcore_axis_name="s")

    @functools.partial(
        pl.kernel,
        mesh=mesh,
        out_type=jax.ShapeDtypeStruct((n_total, dim), jnp.float32),
        scratch_types=[
            pltpu.VMEM((nsteps, _ROWS_PER_DMA), jnp.int32),
            pltpu.VMEM((2, _CHUNK, dim), jnp.float32),
            pltpu.SemaphoreType.DMA,
        ],
        compiler_params=pltpu.CompilerParams(use_tc_tiling_on_sc=False),
    )
    def gather_kernel(idx_hbm, table_hbm, out_hbm, idx_v, rows_v, sem):
        wid = lax.axis_index("s") * 2 + lax.axis_index("c")
        base = wid * per_worker
        pltpu.sync_copy(idx_hbm.at[wid], idx_v)

        def fire(g, b):
            for j in range(_SUBS):
                pltpu.async_copy(
                    table_hbm.at[idx_v.at[g * _SUBS + j]],
                    rows_v.at[b, pl.ds(j * _ROWS_PER_DMA, _ROWS_PER_DMA)],
                    sem,
                )

        def drain_and_scatter(g, b):
            for j in range(_SUBS):
                pltpu.make_async_copy(
                    table_hbm.at[idx_v.at[g * _SUBS + j]],
                    rows_v.at[b, pl.ds(j * _ROWS_PER_DMA, _ROWS_PER_DMA)],
                    sem,
                ).wait()
            pltpu.sync_copy(
                rows_v.at[b], out_hbm.at[pl.ds(base + g * _CHUNK, _CHUNK)]
            )

        fire(0, 0)

        def body(g, carry):
            fire(g + 1, lax.rem(g + 1, 2))
            drain_and_scatter(g, lax.rem(g, 2))
            return carry

        lax.fori_loop(0, nchunks - 1, body, 0)
        drain_and_scatter(nchunks - 1, (nchunks - 1) % 2)

    return gather_kernel


_TR_BLK = 8192


@functools.lru_cache(maxsize=None)
def _make_transpose(v: int, dim: int):
    nblk = (v + _TR_BLK - 1) // _TR_BLK

    fold = 128 // dim

    def _sel(k):
        rows = lax.broadcasted_iota(jnp.int32, (dim, 128), 0)
        cols = lax.broadcasted_iota(jnp.int32, (dim, 128), 1)
        return (cols == k * dim + rows).astype(jnp.float32)

    def body(x_ref, o_ref):
        x3 = x_ref[...].T.reshape(_TR_BLK // fold, fold, dim)
        y = jnp.zeros((_TR_BLK // fold, 128), jnp.float32)
        for k in range(fold):
            y = y + jnp.dot(x3[:, k, :], _sel(k),
                            preferred_element_type=jnp.float32)
        o_ref[...] = y.reshape(-1)

    return pl.pallas_call(
        body,
        grid=(nblk,),
        in_specs=[pl.BlockSpec((dim, _TR_BLK), lambda i: (0, i))],
        out_specs=pl.BlockSpec((_TR_BLK * dim,), lambda i: (i,)),
        out_shape=jax.ShapeDtypeStruct((v * dim,), jnp.float32),
    )


def kernel(indices, table):
    shape = indices.shape
    v, dim = table.shape
    table_rm = _make_transpose(v, dim)(table.T).reshape(v, dim)
    flat = indices.reshape(-1).astype(jnp.int32)
    n = flat.shape[0]
    group = _NUM_WORKERS * _CHUNK
    n_pad = ((n + group - 1) // group) * group
    if n_pad != n:
        flat = jnp.concatenate([flat, jnp.zeros((n_pad - n,), jnp.int32)])
    idx3 = flat.reshape(_NUM_WORKERS, -1, _ROWS_PER_DMA)
    out = _make_gather(n_pad, dim)(idx3, table_rm)
    if n_pad != n:
        out = out[:n]
    return out.reshape(*shape, dim)


# speedup vs baseline: 2.0962x; 1.0944x over previous
import functools

import jax
import jax.numpy as jnp
from jax import lax
from jax.experimental import pallas as pl
from jax.experimental.pallas import tpu as pltpu
from jax.experimental.pallas import tpu_sc as plsc

_NUM_WORKERS = 32
_ROWS_PER_DMA = 128
_SUBS = 8
_CHUNK = _SUBS * _ROWS_PER_DMA


@functools.lru_cache(maxsize=None)
def _make_gather(n_total: int, dim: int):
    assert n_total % (_NUM_WORKERS * _CHUNK) == 0
    per_worker = n_total // _NUM_WORKERS
    nchunks = per_worker // _CHUNK
    nsteps = per_worker // _ROWS_PER_DMA

    mesh = plsc.VectorSubcoreMesh(core_axis_name="c", subcore_axis_name="s")

    @functools.partial(
        pl.kernel,
        mesh=mesh,
        out_type=jax.ShapeDtypeStruct((n_total, dim), jnp.float32),
        scratch_types=[
            pltpu.VMEM((nsteps, _ROWS_PER_DMA), jnp.int32),
            pltpu.VMEM((2, _CHUNK, dim), jnp.float32),
            pltpu.SemaphoreType.DMA,
        ],
        compiler_params=pltpu.CompilerParams(use_tc_tiling_on_sc=False),
    )
    def gather_kernel(idx_hbm, table_hbm, out_hbm, idx_v, rows_v, sem):
        wid = lax.axis_index("s") * 2 + lax.axis_index("c")
        base = wid * per_worker
        pltpu.sync_copy(idx_hbm.at[wid], idx_v)

        def fire(g, b):
            for j in range(_SUBS):
                pltpu.async_copy(
                    table_hbm.at[idx_v.at[g * _SUBS + j]],
                    rows_v.at[b, pl.ds(j * _ROWS_PER_DMA, _ROWS_PER_DMA)],
                    sem,
                )

        def drain_and_scatter(g, b):
            for j in range(_SUBS):
                pltpu.make_async_copy(
                    table_hbm.at[idx_v.at[g * _SUBS + j]],
                    rows_v.at[b, pl.ds(j * _ROWS_PER_DMA, _ROWS_PER_DMA)],
                    sem,
                ).wait()
            pltpu.sync_copy(
                rows_v.at[b], out_hbm.at[pl.ds(base + g * _CHUNK, _CHUNK)]
            )

        fire(0, 0)

        def body(g, carry):
            fire(g + 1, lax.rem(g + 1, 2))
            drain_and_scatter(g, lax.rem(g, 2))
            return carry

        lax.fori_loop(0, nchunks - 1, body, 0)
        drain_and_scatter(nchunks - 1, (nchunks - 1) % 2)

    return gather_kernel


_TR_BLK = 8192


@functools.lru_cache(maxsize=None)
def _make_transpose(v: int, dim: int):
    nblk = (v + _TR_BLK - 1) // _TR_BLK

    fold = 128 // dim

    def _sel(k):
        rows = lax.broadcasted_iota(jnp.int32, (dim, 128), 0)
        cols = lax.broadcasted_iota(jnp.int32, (dim, 128), 1)
        return (cols == k * dim + rows).astype(jnp.float32)

    def body(x_ref, o_ref):
        x3 = x_ref[...].T.reshape(_TR_BLK // fold, fold, dim)
        y = jnp.zeros((_TR_BLK // fold, 128), jnp.float32)
        for k in range(fold):
            y = y + jnp.dot(x3[:, k, :], _sel(k),
                            preferred_element_type=jnp.float32)
        o_ref[...] = y.reshape(-1)

    return pl.pallas_call(
        body,
        grid=(nblk,),
        in_specs=[pl.BlockSpec((dim, _TR_BLK), lambda i: (0, i))],
        out_specs=pl.BlockSpec((_TR_BLK * dim,), lambda i: (i,)),
        out_shape=jax.ShapeDtypeStruct((v * dim,), jnp.float32),
    )


def kernel(indices, table):
    ni, nj = indices.shape
    v, dim = table.shape
    table_rm = _make_transpose(v, dim)(table.T).reshape(v, dim)
    flat = indices.T.reshape(-1).astype(jnp.int32)
    n = flat.shape[0]
    group = _NUM_WORKERS * _CHUNK
    n_pad = ((n + group - 1) // group) * group
    if n_pad != n:
        flat = jnp.concatenate([flat, jnp.zeros((n_pad - n,), jnp.int32)])
    idx3 = flat.reshape(_NUM_WORKERS, -1, _ROWS_PER_DMA)
    out = _make_gather(n_pad, dim)(idx3, table_rm)
    if n_pad != n:
        out = out[:n]
    return out.reshape(nj, ni, dim).transpose(1, 0, 2)
